# baseline (speedup 1.0000x reference)
import jax
import jax.numpy as jnp
from jax import lax
from jax.experimental import pallas as pl
from jax.experimental.pallas import tpu as pltpu
from jax.experimental.pallas import tpu_sc as plsc

N_ROWS = 100000
B = 16384
NC, NS, L = 2, 16, 16
NW = NC * NS
BPW = B // NW
NCHUNK = BPW // 128
NGROUP = BPW // L

_A_COEF = (1.0, -1.0 / 6.0, 1.0 / 120.0, -1.0 / 5040.0,
           1.0 / 362880.0, -1.0 / 39916800.0)
_B_COEF = (0.5, -1.0 / 24.0, 1.0 / 720.0, -1.0 / 40320.0,
           1.0 / 3628800.0, -1.0 / 479001600.0)


def _poly(t, coef):
    acc = jnp.full((L,), coef[-1], jnp.float32)
    for c in reversed(coef[:-1]):
        acc = acc * t + c
    return acc


def _body(idx_hbm, *refs):
    rin = refs[0:9]
    win = refs[9:12]
    sin_ = refs[12:14]
    rout = refs[14:23]
    sout = refs[23:25]
    idx_v = refs[25]
    rcol = refs[26:35]
    wcol = refs[35:38]
    scol = refs[38:40]
    ocol = refs[40:49]
    sem = refs[49]

    wid = lax.axis_index("s") * NC + lax.axis_index("c")
    base = wid * BPW

    pltpu.sync_copy(idx_hbm.at[pl.ds(wid * NCHUNK, NCHUNK)], idx_v)

    copies = []
    for c in range(NCHUNK):
        sl = pl.ds(c * 128, 128)
        ic = idx_v.at[c]
        for d in range(9):
            copies.append(pltpu.async_copy(rin[d].at[ic], rcol[d].at[sl], sem))
        for d in range(3):
            copies.append(pltpu.async_copy(win[d].at[ic], wcol[d].at[sl], sem))
        for d in range(2):
            copies.append(pltpu.async_copy(sin_[d].at[ic], scol[d].at[sl], sem))
    for cp in copies:
        cp.wait()

    def group(g, carry):
        sl = pl.ds(g * L, L)
        w0 = wcol[0][sl]
        w1 = wcol[1][sl]
        w2 = wcol[2][sl]
        r = [rcol[d][sl] for d in range(9)]

        w00, w11, w22 = w0 * w0, w1 * w1, w2 * w2
        t = w00 + w11 + w22
        A = _poly(t, _A_COEF)
        Bc = _poly(t, _B_COEF)
        w01, w02, w12 = w0 * w1, w0 * w2, w1 * w2
        a0, a1, a2 = A * w0, A * w1, A * w2
        p00 = 1.0 - Bc * (w11 + w22)
        p01 = Bc * w01 - a2
        p02 = Bc * w02 + a1
        p10 = Bc * w01 + a2
        p11 = 1.0 - Bc * (w00 + w22)
        p12 = Bc * w12 - a0
        p20 = Bc * w02 - a1
        p21 = Bc * w12 + a0
        p22 = 1.0 - Bc * (w00 + w11)
        p = ((p00, p01, p02), (p10, p11, p12), (p20, p21, p22))
        for i in range(3):
            for j in range(3):
                acc = p[i][0] * r[0 * 3 + j]
                acc = acc + p[i][1] * r[1 * 3 + j]
                acc = acc + p[i][2] * r[2 * 3 + j]
                ocol[i * 3 + j][sl] = acc
        return carry

    lax.fori_loop(0, NGROUP, group, 0, unroll=False)

    for d in range(9):
        pltpu.sync_copy(ocol[d], rout[d].at[pl.ds(base, BPW)])
    for d in range(2):
        pltpu.sync_copy(scol[d], sout[d].at[pl.ds(base, BPW)])


@jax.jit
def _run(idx2d, *planes):
    mesh = plsc.VectorSubcoreMesh(core_axis_name="c", subcore_axis_name="s",
                                  num_cores=NC, num_subcores=NS)
    scratch = [pltpu.VMEM((NCHUNK, 128), jnp.int32)]
    scratch += [pltpu.VMEM((BPW,), jnp.float32)] * 23
    scratch += [pltpu.SemaphoreType.DMA]
    f = pl.kernel(
        _body,
        out_type=tuple([jax.ShapeDtypeStruct((B,), jnp.float32)] * 11),
        mesh=mesh,
        scratch_types=scratch,
    )
    return f(idx2d, *planes)


def kernel(idx, rotations, perturbations_w, shifts):
    idx2d = idx.astype(jnp.int32).reshape(B // 128, 128)
    rot_t = jnp.transpose(rotations, (1, 2, 0)).reshape(9, N_ROWS)
    pw_t = perturbations_w.T
    sh_t = shifts.T
    planes = [rot_t[d] for d in range(9)]
    planes += [pw_t[c] for c in range(3)]
    planes += [sh_t[c] for c in range(2)]
    outs = _run(idx2d, *planes)
    rots = jnp.stack(outs[0:9], axis=-1).reshape(B, 3, 3)
    sh = jnp.stack(outs[9:11], axis=-1)
    return rots, sh


# speedup vs baseline: 1.3485x; 1.3485x over previous
import jax
import jax.numpy as jnp
from jax import lax
from jax.experimental import pallas as pl
from jax.experimental.pallas import tpu as pltpu
from jax.experimental.pallas import tpu_sc as plsc

N_ROWS = 100000
B = 16384
NC, NS, L = 2, 16, 16
NW = NC * NS
BPW = B // NW
NCHUNK = BPW // 128
NGROUP = BPW // L

_A_COEF = (1.0, -1.0 / 6.0, 1.0 / 120.0, -1.0 / 5040.0,
           1.0 / 362880.0, -1.0 / 39916800.0)
_B_COEF = (0.5, -1.0 / 24.0, 1.0 / 720.0, -1.0 / 40320.0,
           1.0 / 3628800.0, -1.0 / 479001600.0)


def _poly(t, coef):
    acc = jnp.full((L,), coef[-1], jnp.float32)
    for c in reversed(coef[:-1]):
        acc = acc * t + c
    return acc


def _body1(idx_hbm, w0_hbm, w1_hbm, w2_hbm, s0_hbm, s1_hbm, *refs):
    pout = refs[0:9]
    sout = refs[9:11]
    idx_v = refs[11]
    wcol = refs[12:15]
    scol = refs[15:17]
    pcol = refs[17:26]
    sem = refs[26]

    wid = lax.axis_index("s") * NC + lax.axis_index("c")
    base = wid * BPW
    pltpu.sync_copy(idx_hbm.at[pl.ds(wid * NCHUNK, NCHUNK)], idx_v)

    win = (w0_hbm, w1_hbm, w2_hbm)
    sin_ = (s0_hbm, s1_hbm)
    copies = []
    for c in range(NCHUNK):
        sl = pl.ds(c * 128, 128)
        ic = idx_v.at[c]
        for d in range(3):
            copies.append(pltpu.async_copy(win[d].at[ic], wcol[d].at[sl], sem))
        for d in range(2):
            copies.append(pltpu.async_copy(sin_[d].at[ic], scol[d].at[sl], sem))
    for cp in copies:
        cp.wait()

    def group(g, carry):
        sl = pl.ds(g * L, L)
        w0 = wcol[0][sl]
        w1 = wcol[1][sl]
        w2 = wcol[2][sl]
        w00, w11, w22 = w0 * w0, w1 * w1, w2 * w2
        t = w00 + w11 + w22
        A = _poly(t, _A_COEF)
        Bc = _poly(t, _B_COEF)
        w01, w02, w12 = w0 * w1, w0 * w2, w1 * w2
        a0, a1, a2 = A * w0, A * w1, A * w2
        pcol[0][sl] = 1.0 - Bc * (w11 + w22)
        pcol[1][sl] = Bc * w01 - a2
        pcol[2][sl] = Bc * w02 + a1
        pcol[3][sl] = Bc * w01 + a2
        pcol[4][sl] = 1.0 - Bc * (w00 + w22)
        pcol[5][sl] = Bc * w12 - a0
        pcol[6][sl] = Bc * w02 - a1
        pcol[7][sl] = Bc * w12 + a0
        pcol[8][sl] = 1.0 - Bc * (w00 + w11)
        return carry

    lax.fori_loop(0, NGROUP, group, 0, unroll=False)

    for d in range(9):
        pltpu.sync_copy(pcol[d], pout[d].at[pl.ds(base, BPW)])
    for d in range(2):
        pltpu.sync_copy(scol[d], sout[d].at[pl.ds(base, BPW)])


def _body2(idx_hbm, *refs):
    rin = refs[0:9]
    pin = refs[9:18]
    rout = refs[18:27]
    idx_v = refs[27]
    rcol = refs[28:37]
    pcol = refs[37:46]
    ocol = refs[46:55]
    sem = refs[55]

    wid = lax.axis_index("s") * NC + lax.axis_index("c")
    base = wid * BPW
    pltpu.sync_copy(idx_hbm.at[pl.ds(wid * NCHUNK, NCHUNK)], idx_v)

    copies = []
    for d in range(9):
        copies.append(pltpu.async_copy(pin[d].at[pl.ds(base, BPW)], pcol[d], sem))
    for c in range(NCHUNK):
        sl = pl.ds(c * 128, 128)
        ic = idx_v.at[c]
        for d in range(9):
            copies.append(pltpu.async_copy(rin[d].at[ic], rcol[d].at[sl], sem))
    for cp in copies:
        cp.wait()

    def group(g, carry):
        sl = pl.ds(g * L, L)
        r = [rcol[d][sl] for d in range(9)]
        p = [pcol[d][sl] for d in range(9)]
        for i in range(3):
            for j in range(3):
                acc = p[i * 3 + 0] * r[0 * 3 + j]
                acc = acc + p[i * 3 + 1] * r[1 * 3 + j]
                acc = acc + p[i * 3 + 2] * r[2 * 3 + j]
                ocol[i * 3 + j][sl] = acc
        return carry

    lax.fori_loop(0, NGROUP, group, 0, unroll=False)

    for d in range(9):
        pltpu.sync_copy(ocol[d], rout[d].at[pl.ds(base, BPW)])


@jax.jit
def _run(idx2d, rplanes, wplanes, splanes):
    mesh = plsc.VectorSubcoreMesh(core_axis_name="c", subcore_axis_name="s",
                                  num_cores=NC, num_subcores=NS)
    f1 = pl.kernel(
        _body1,
        out_type=tuple([jax.ShapeDtypeStruct((B,), jnp.float32)] * 11),
        mesh=mesh,
        scratch_types=[pltpu.VMEM((NCHUNK, 128), jnp.int32)]
        + [pltpu.VMEM((BPW,), jnp.float32)] * 14
        + [pltpu.SemaphoreType.DMA],
    )
    outs1 = f1(idx2d, *wplanes, *splanes)
    pplanes, shout = outs1[0:9], outs1[9:11]

    f2 = pl.kernel(
        _body2,
        out_type=tuple([jax.ShapeDtypeStruct((B,), jnp.float32)] * 9),
        mesh=mesh,
        scratch_types=[pltpu.VMEM((NCHUNK, 128), jnp.int32)]
        + [pltpu.VMEM((BPW,), jnp.float32)] * 27
        + [pltpu.SemaphoreType.DMA],
    )
    routs = f2(idx2d, *rplanes, *pplanes)
    return routs, shout


def kernel(idx, rotations, perturbations_w, shifts):
    idx2d = idx.astype(jnp.int32).reshape(B // 128, 128)
    rplanes = [rotations[:, i, j] for i in range(3) for j in range(3)]
    wplanes = [perturbations_w[:, c] for c in range(3)]
    splanes = [shifts[:, c] for c in range(2)]
    routs, shout = _run(idx2d, rplanes, wplanes, splanes)
    rots = jnp.stack(routs, axis=-1).reshape(B, 3, 3)
    sh = jnp.stack(shout, axis=-1)
    return rots, sh
